# Initial kernel scaffold; baseline (speedup 1.0000x reference)
import jax
import jax.numpy as jnp
from jax.experimental import pallas as pl


def kernel(features, mask, centroids):
    raise NotImplementedError("write your pallas kernel here")


# speedup vs baseline: 1.9294x; 1.9294x over previous
import functools

import jax
import jax.numpy as jnp
from jax.experimental import pallas as pl
from jax.experimental.pallas import tpu as pltpu


def _bow_kernel(feat_ref, maskf_ref, cent_ref, hist_ref, *, blk, num_bags):
    i = pl.program_id(1)
    feat = feat_ref[0]
    cent = cent_ref[...]
    scores = jax.lax.dot_general(
        feat, cent, (((1,), (1,)), ((), ())),
        preferred_element_type=jnp.float32)
    rnorm = jnp.sum(feat * feat, axis=1, keepdims=True)
    cnorm = jnp.sum(cent * cent, axis=1)
    dists = rnorm - 2.0 * scores + cnorm[None, :]
    nearest = jnp.argmin(dists, axis=1)
    valid = 1.0 - maskf_ref[0, 0]
    onehot = (nearest[:, None]
              == jax.lax.broadcasted_iota(jnp.int32, (blk, num_bags), 1))
    contrib = jnp.sum(onehot.astype(jnp.float32) * valid[:, None], axis=0)

    @pl.when(i == 0)
    def _init():
        hist_ref[...] = contrib[None, None, :]

    @pl.when(i != 0)
    def _acc():
        hist_ref[...] += contrib[None, None, :]


def kernel(features, mask, centroids):
    nb, nc, d = features.shape
    num_bags = centroids.shape[0]
    blk = 512
    num_blk = nc // blk
    maskf = mask.astype(jnp.float32).reshape(nb * num_blk, 1, blk)

    grid = (nb, num_blk)
    hist = pl.pallas_call(
        functools.partial(_bow_kernel, blk=blk, num_bags=num_bags),
        grid=grid,
        in_specs=[
            pl.BlockSpec((1, blk, d), lambda b, i: (b, i, 0)),
            pl.BlockSpec((1, 1, blk), lambda b, i, nbk=num_blk: (b * nbk + i, 0, 0)),
            pl.BlockSpec((num_bags, d), lambda b, i: (0, 0)),
        ],
        out_specs=pl.BlockSpec((1, 1, num_bags), lambda b, i: (b, 0, 0)),
        out_shape=jax.ShapeDtypeStruct((nb, 1, num_bags), jnp.float32),
        compiler_params=pltpu.CompilerParams(
            dimension_semantics=("arbitrary", "arbitrary")),
    )(features, maskf, centroids)
    return hist.reshape(nb, num_bags)
